# Initial kernel scaffold; baseline (speedup 1.0000x reference)
import jax
import jax.numpy as jnp
from jax.experimental import pallas as pl


def kernel(x, W1, b1, W2, b2):
    raise NotImplementedError("write your pallas kernel here")


# speedup vs baseline: 2.2988x; 2.2988x over previous
import functools

import jax
import jax.numpy as jnp
from jax.experimental import pallas as pl

_NUM_TOKENS = 32768
_INPUT_DIM = 768
_HIDDEN = 128
_NUM_EXPERTS = 64
_BT = 1024


def _router_block(x_ref, w1_ref, b1_ref, w2_ref, b2_ref,
                  idx_ref, p_ref, u_ref):
    h = jnp.dot(x_ref[:], w1_ref[:], preferred_element_type=jnp.float32)
    h = jnp.maximum(h + b1_ref[:], 0.0)
    logits = jnp.dot(h, w2_ref[:], preferred_element_type=jnp.float32)
    logits = logits + b2_ref[:]

    m1 = jnp.max(logits, axis=1, keepdims=True)
    e = jnp.exp(logits - m1)
    s = jnp.sum(e, axis=1, keepdims=True)

    iota = jax.lax.broadcasted_iota(jnp.int32, logits.shape, 1)
    sentinel = jnp.int32(_NUM_EXPERTS)
    i1 = jnp.min(jnp.where(logits == m1, iota, sentinel), axis=1,
                 keepdims=True)
    masked = jnp.where(iota == i1, -jnp.inf, logits)
    m2 = jnp.max(masked, axis=1, keepdims=True)
    i2 = jnp.min(jnp.where(masked == m2, iota, sentinel), axis=1,
                 keepdims=True)

    idx_ref[:] = jnp.concatenate([i1, i2], axis=1)
    p_ref[:] = jnp.concatenate([1.0 / s, jnp.exp(m2 - m1) / s], axis=1)
    u_ref[:] = jnp.zeros_like(u_ref)


@jax.jit
def kernel(x, W1, b1, W2, b2):
    grid = (_NUM_TOKENS // _BT,)
    out = pl.pallas_call(
        _router_block,
        grid=grid,
        in_specs=[
            pl.BlockSpec((_BT, _INPUT_DIM), lambda i: (i, 0)),
            pl.BlockSpec((_INPUT_DIM, _HIDDEN), lambda i: (0, 0)),
            pl.BlockSpec((1, _HIDDEN), lambda i: (0, 0)),
            pl.BlockSpec((_HIDDEN, _NUM_EXPERTS), lambda i: (0, 0)),
            pl.BlockSpec((1, _NUM_EXPERTS), lambda i: (0, 0)),
        ],
        out_specs=[
            pl.BlockSpec((_BT, 2), lambda i: (i, 0)),
            pl.BlockSpec((_BT, 2), lambda i: (i, 0)),
            pl.BlockSpec((_BT, 1), lambda i: (i, 0)),
        ],
        out_shape=[
            jax.ShapeDtypeStruct((_NUM_TOKENS, 2), jnp.int32),
            jax.ShapeDtypeStruct((_NUM_TOKENS, 2), jnp.float32),
            jax.ShapeDtypeStruct((_NUM_TOKENS, 1), jnp.float32),
        ],
    )(x, W1, b1.reshape(1, _HIDDEN), W2, b2.reshape(1, _NUM_EXPERTS))
    top_k_indices, top_k_probs, uncertainty = out
    return (top_k_indices, top_k_probs, uncertainty.reshape(_NUM_TOKENS))


# speedup vs baseline: 2.5469x; 1.1079x over previous
import functools

import jax
import jax.numpy as jnp
from jax.experimental import pallas as pl

_NUM_TOKENS = 32768
_INPUT_DIM = 768
_HIDDEN = 128
_NUM_EXPERTS = 64
_BT = 2048


def _router_block(x_ref, w1_ref, b1_ref, w2_ref, b2_ref,
                  idx_ref, p_ref, u_ref):
    h = jnp.dot(x_ref[:], w1_ref[:], preferred_element_type=jnp.float32)
    h = jnp.maximum(h + b1_ref[:], 0.0)
    logits = jnp.dot(h, w2_ref[:], preferred_element_type=jnp.float32)
    logits = logits + b2_ref[:]

    m1 = jnp.max(logits, axis=1, keepdims=True)
    e = jnp.exp(logits - m1)
    s = jnp.sum(e, axis=1, keepdims=True)

    iota = jax.lax.broadcasted_iota(jnp.int32, logits.shape, 1)
    sentinel = jnp.int32(_NUM_EXPERTS)
    i1 = jnp.min(jnp.where(logits == m1, iota, sentinel), axis=1,
                 keepdims=True)
    masked = jnp.where(iota == i1, -jnp.inf, logits)
    m2 = jnp.max(masked, axis=1, keepdims=True)
    i2 = jnp.min(jnp.where(masked == m2, iota, sentinel), axis=1,
                 keepdims=True)

    idx_ref[:] = jnp.concatenate([i1, i2], axis=1)
    p_ref[:] = jnp.concatenate([1.0 / s, jnp.exp(m2 - m1) / s], axis=1)
    u_ref[:] = jnp.zeros_like(u_ref)


@jax.jit
def kernel(x, W1, b1, W2, b2):
    grid = (_NUM_TOKENS // _BT,)
    out = pl.pallas_call(
        _router_block,
        grid=grid,
        in_specs=[
            pl.BlockSpec((_BT, _INPUT_DIM), lambda i: (i, 0)),
            pl.BlockSpec((_INPUT_DIM, _HIDDEN), lambda i: (0, 0)),
            pl.BlockSpec((1, _HIDDEN), lambda i: (0, 0)),
            pl.BlockSpec((_HIDDEN, _NUM_EXPERTS), lambda i: (0, 0)),
            pl.BlockSpec((1, _NUM_EXPERTS), lambda i: (0, 0)),
        ],
        out_specs=[
            pl.BlockSpec((_BT, 2), lambda i: (i, 0)),
            pl.BlockSpec((_BT, 2), lambda i: (i, 0)),
            pl.BlockSpec((_BT, 1), lambda i: (i, 0)),
        ],
        out_shape=[
            jax.ShapeDtypeStruct((_NUM_TOKENS, 2), jnp.int32),
            jax.ShapeDtypeStruct((_NUM_TOKENS, 2), jnp.float32),
            jax.ShapeDtypeStruct((_NUM_TOKENS, 1), jnp.float32),
        ],
    )(x, W1, b1.reshape(1, _HIDDEN), W2, b2.reshape(1, _NUM_EXPERTS))
    top_k_indices, top_k_probs, uncertainty = out
    return (top_k_indices, top_k_probs, uncertainty.reshape(_NUM_TOKENS))
